# Initial kernel scaffold; baseline (speedup 1.0000x reference)
import functools

import jax
import jax.numpy as jnp
from jax import lax
from jax.experimental import pallas as pl
from jax.experimental.pallas import tpu as pltpu
from jax.experimental.pallas import tpu_sc as plsc

N_NODES = 10000
N_EDGES = 320000
D = 128

NC = 2
NS = 16
LANES = 16

E_PER_TILE = N_EDGES // (NC * NS)
CHUNK = 128
N_FULL = E_PER_TILE // CHUNK
REM = E_PER_TILE - N_FULL * CHUNK

ROWS_PER_TILE = N_NODES // NS
ZCHUNK = 125


def _relu_add_rows(gbuf, ebuf, n_rows):
    def row(r, carry):
        for c in range(D // LANES):
            s = pl.ds(c * LANES, LANES)
            ebuf[r, s] = jnp.maximum(gbuf[r, s] + ebuf[r, s], 0.0)
        return carry

    lax.fori_loop(0, n_rows, row, 0, unroll=2)


def _sc_body(nodes_hbm, src_hbm, dst_hbm, edges_hbm, out_hbm,
             idx_s, idx_d, gbuf, ebuf, idx_s_r, idx_d_r, gbuf_r, ebuf_r,
             acc, sem):
    cid = lax.axis_index("c")
    sid = lax.axis_index("s")
    wid = cid * NS + sid

    def zrow(r, carry):
        for c in range(D // LANES):
            gbuf[r, pl.ds(c * LANES, LANES)] = jnp.zeros((LANES,), jnp.float32)
        return carry

    lax.fori_loop(0, ZCHUNK, zrow, 0)
    row0 = sid * ROWS_PER_TILE
    for i in range(ROWS_PER_TILE // ZCHUNK):
        pltpu.sync_copy(gbuf.at[pl.ds(0, ZCHUNK)],
                        acc.at[pl.ds(row0 + i * ZCHUNK, ZCHUNK)])
    plsc.subcore_barrier()

    base = wid * E_PER_TILE

    def chunk(k, carry):
        e0 = base + k * CHUNK
        pltpu.sync_copy(src_hbm.at[pl.ds(e0, CHUNK)], idx_s)
        pltpu.sync_copy(dst_hbm.at[pl.ds(e0, CHUNK)], idx_d)
        g = pltpu.async_copy(nodes_hbm.at[idx_s], gbuf, sem)
        pltpu.sync_copy(edges_hbm.at[pl.ds(e0, CHUNK)], ebuf)
        g.wait()
        _relu_add_rows(gbuf, ebuf, CHUNK)
        pltpu.sync_copy(ebuf, acc.at[idx_d], add=True)
        return carry

    lax.fori_loop(0, N_FULL, chunk, 0)

    e0 = base + N_FULL * CHUNK
    pltpu.sync_copy(src_hbm.at[pl.ds(e0, REM)], idx_s_r)
    pltpu.sync_copy(dst_hbm.at[pl.ds(e0, REM)], idx_d_r)
    g = pltpu.async_copy(nodes_hbm.at[idx_s_r], gbuf_r, sem)
    pltpu.sync_copy(edges_hbm.at[pl.ds(e0, REM)], ebuf_r)
    g.wait()
    _relu_add_rows(gbuf_r, ebuf_r, REM)
    pltpu.sync_copy(ebuf_r, acc.at[idx_d_r], add=True)

    plsc.subcore_barrier()
    out_base = cid * N_NODES + sid * ROWS_PER_TILE
    for i in range(ROWS_PER_TILE // ZCHUNK):
        pltpu.sync_copy(acc.at[pl.ds(row0 + i * ZCHUNK, ZCHUNK)],
                        out_hbm.at[pl.ds(out_base + i * ZCHUNK, ZCHUNK)])


@jax.jit
def _sc_partials(nodes, src, dst, edges):
    mesh = plsc.VectorSubcoreMesh(core_axis_name="c", subcore_axis_name="s")
    return pl.kernel(
        _sc_body,
        out_type=jax.ShapeDtypeStruct((NC * N_NODES, D), jnp.float32),
        mesh=mesh,
        scratch_types=[
            pltpu.VMEM((CHUNK,), jnp.int32),
            pltpu.VMEM((CHUNK,), jnp.int32),
            pltpu.VMEM((CHUNK, D), jnp.float32),
            pltpu.VMEM((CHUNK, D), jnp.float32),
            pltpu.VMEM((REM,), jnp.int32),
            pltpu.VMEM((REM,), jnp.int32),
            pltpu.VMEM((REM, D), jnp.float32),
            pltpu.VMEM((REM, D), jnp.float32),
            pltpu.VMEM_SHARED((N_NODES, D), jnp.float32),
            pltpu.SemaphoreType.DMA,
        ],
    )(nodes, src, dst, edges)


def _combine_body(eps_ref, nodes_ref, p0_ref, p1_ref, out_ref):
    scale = 1.0 + eps_ref[0, 0]
    out_ref[...] = scale * nodes_ref[...] + p0_ref[...] + p1_ref[...]


@jax.jit
def _combine(nodes, partials, eps):
    blk = 1000
    grid = N_NODES // blk
    return pl.pallas_call(
        _combine_body,
        grid=(grid,),
        in_specs=[
            pl.BlockSpec(memory_space=pltpu.SMEM),
            pl.BlockSpec((blk, D), lambda i: (i, 0)),
            pl.BlockSpec((blk, D), lambda i: (i, 0)),
            pl.BlockSpec((blk, D), lambda i: (i + grid, 0)),
        ],
        out_specs=pl.BlockSpec((blk, D), lambda i: (i, 0)),
        out_shape=jax.ShapeDtypeStruct((N_NODES, D), jnp.float32),
    )(eps.reshape(1, 1), nodes, partials, partials)


def kernel(nodes, edge_index, edges, eps):
    src = edge_index[1].astype(jnp.int32)
    dst = edge_index[0].astype(jnp.int32)
    partials = _sc_partials(nodes, src, dst, edges)
    return _combine(nodes, partials, eps.astype(jnp.float32))


# speedup vs baseline: 2.6150x; 2.6150x over previous
import functools

import jax
import jax.numpy as jnp
from jax import lax
from jax.experimental import pallas as pl
from jax.experimental.pallas import tpu as pltpu
from jax.experimental.pallas import tpu_sc as plsc

N_NODES = 10000
N_EDGES = 320000
D = 128

NC = 2
NS = 16
LANES = 16

E_PER_TILE = N_EDGES // (NC * NS)
CHUNK = 128
N_FULL = E_PER_TILE // CHUNK
REM = E_PER_TILE - N_FULL * CHUNK

ROWS_PER_TILE = 624
LAST_EXTRA = N_NODES - NS * ROWS_PER_TILE
ZCHUNK = 104


def _relu_add_rows(gbuf, ebuf, n_rows):
    def row(r, carry):
        for c in range(D // LANES):
            s = pl.ds(c * LANES, LANES)
            ebuf[r, s] = jnp.maximum(gbuf[r, s] + ebuf[r, s], 0.0)
        return carry

    lax.fori_loop(0, n_rows, row, 0, unroll=2)


def _sc_body(nodes_hbm, src_hbm, dst_hbm, edges_hbm, out_hbm,
             idx_s, idx_d, gbuf, ebuf, idx_s_r, idx_d_r, gbuf_r, ebuf_r,
             acc, sem):
    cid = lax.axis_index("c")
    sid = lax.axis_index("s")
    wid = cid * NS + sid

    def zrow(r, carry):
        for c in range(D // LANES):
            gbuf[r, pl.ds(c * LANES, LANES)] = jnp.zeros((LANES,), jnp.float32)
        return carry

    lax.fori_loop(0, ZCHUNK, zrow, 0)
    row0 = sid * ROWS_PER_TILE
    for i in range(ROWS_PER_TILE // ZCHUNK):
        pltpu.sync_copy(gbuf.at[pl.ds(0, ZCHUNK)],
                        acc.at[pl.ds(row0 + i * ZCHUNK, ZCHUNK)])

    @pl.when(sid == NS - 1)
    def _zero_tail():
        pltpu.sync_copy(gbuf.at[pl.ds(0, LAST_EXTRA)],
                        acc.at[pl.ds(NS * ROWS_PER_TILE, LAST_EXTRA)])

    plsc.subcore_barrier()

    base = wid * E_PER_TILE

    def chunk(k, carry):
        e0 = base + k * CHUNK
        pltpu.sync_copy(src_hbm.at[pl.ds(e0, CHUNK)], idx_s)
        pltpu.sync_copy(dst_hbm.at[pl.ds(e0, CHUNK)], idx_d)
        g = pltpu.async_copy(nodes_hbm.at[idx_s], gbuf, sem)
        pltpu.sync_copy(edges_hbm.at[pl.ds(e0, CHUNK)], ebuf)
        g.wait()
        _relu_add_rows(gbuf, ebuf, CHUNK)
        pltpu.sync_copy(ebuf, acc.at[idx_d], add=True)
        return carry

    lax.fori_loop(0, N_FULL, chunk, 0)

    e0 = base + N_FULL * CHUNK
    pltpu.sync_copy(src_hbm.at[pl.ds(e0, REM)], idx_s_r)
    pltpu.sync_copy(dst_hbm.at[pl.ds(e0, REM)], idx_d_r)
    g = pltpu.async_copy(nodes_hbm.at[idx_s_r], gbuf_r, sem)
    pltpu.sync_copy(edges_hbm.at[pl.ds(e0, REM)], ebuf_r)
    g.wait()
    _relu_add_rows(gbuf_r, ebuf_r, REM)
    pltpu.sync_copy(ebuf_r, acc.at[idx_d_r], add=True)

    plsc.subcore_barrier()
    out_base = cid * N_NODES + sid * ROWS_PER_TILE
    pltpu.sync_copy(acc.at[pl.ds(row0, ROWS_PER_TILE)],
                    out_hbm.at[pl.ds(out_base, ROWS_PER_TILE)])

    @pl.when(sid == NS - 1)
    def _write_tail():
        pltpu.sync_copy(acc.at[pl.ds(NS * ROWS_PER_TILE, LAST_EXTRA)],
                        out_hbm.at[pl.ds(cid * N_NODES + NS * ROWS_PER_TILE,
                                         LAST_EXTRA)])


@jax.jit
def _sc_partials(nodes, src, dst, edges):
    mesh = plsc.VectorSubcoreMesh(core_axis_name="c", subcore_axis_name="s")
    return pl.kernel(
        _sc_body,
        out_type=jax.ShapeDtypeStruct((NC * N_NODES, D), jnp.float32),
        mesh=mesh,
        scratch_types=[
            pltpu.VMEM((CHUNK,), jnp.int32),
            pltpu.VMEM((CHUNK,), jnp.int32),
            pltpu.VMEM((CHUNK, D), jnp.float32),
            pltpu.VMEM((CHUNK, D), jnp.float32),
            pltpu.VMEM((REM,), jnp.int32),
            pltpu.VMEM((REM,), jnp.int32),
            pltpu.VMEM((REM, D), jnp.float32),
            pltpu.VMEM((REM, D), jnp.float32),
            pltpu.VMEM_SHARED((N_NODES, D), jnp.float32),
            pltpu.SemaphoreType.DMA,
        ],
    )(nodes, src, dst, edges)


def _combine_body(eps_ref, nodes_ref, p0_ref, p1_ref, out_ref):
    scale = 1.0 + eps_ref[0, 0]
    out_ref[...] = scale * nodes_ref[...] + p0_ref[...] + p1_ref[...]


@jax.jit
def _combine(nodes, partials, eps):
    blk = 1000
    grid = N_NODES // blk
    return pl.pallas_call(
        _combine_body,
        grid=(grid,),
        in_specs=[
            pl.BlockSpec(memory_space=pltpu.SMEM),
            pl.BlockSpec((blk, D), lambda i: (i, 0)),
            pl.BlockSpec((blk, D), lambda i: (i, 0)),
            pl.BlockSpec((blk, D), lambda i: (i + grid, 0)),
        ],
        out_specs=pl.BlockSpec((blk, D), lambda i: (i, 0)),
        out_shape=jax.ShapeDtypeStruct((N_NODES, D), jnp.float32),
    )(eps.reshape(1, 1), nodes, partials, partials)


def kernel(nodes, edge_index, edges, eps):
    src = edge_index[1].astype(jnp.int32)
    dst = edge_index[0].astype(jnp.int32)
    partials = _sc_partials(nodes, src, dst, edges)
    return _combine(nodes, partials, eps.astype(jnp.float32))
